# Initial kernel scaffold; baseline (speedup 1.0000x reference)
import jax
import jax.numpy as jnp
from jax.experimental import pallas as pl


def kernel(x, gate_w, lb_bias, w1, w2, shared_fc, shared_proj):
    raise NotImplementedError("write your pallas kernel here")


# speedup vs baseline: 1.4344x; 1.4344x over previous
import functools

import jax
import jax.numpy as jnp
from jax import lax
from jax.experimental import pallas as pl
from jax.experimental.pallas import tpu as pltpu

_INTERPRET = False

E = 8
TOK_BLK = 256


def _dense_body(x_ref, gw_ref, lb_ref, w1_ref, w2_ref, sfc_ref, spr_ref, out_ref):
    xb = x_ref[...]
    xb16 = xb.astype(jnp.bfloat16)

    logits = lax.dot_general(
        xb, gw_ref[...], (((1,), (1,)), ((), ())),
        preferred_element_type=jnp.float32,
    )
    sel = logits + lb_ref[...]
    iota = lax.broadcasted_iota(jnp.int32, sel.shape, 1)
    neg = jnp.float32(-1e30)

    m1 = jnp.max(sel, axis=1, keepdims=True)
    idx1 = jnp.min(jnp.where(sel >= m1, iota, E), axis=1, keepdims=True)
    pick1 = iota == idx1
    s1 = jnp.sum(jnp.where(pick1, logits, 0.0), axis=1, keepdims=True)

    sel2 = jnp.where(pick1, neg, sel)
    m2 = jnp.max(sel2, axis=1, keepdims=True)
    idx2 = jnp.min(jnp.where(sel2 >= m2, iota, E), axis=1, keepdims=True)
    pick2 = iota == idx2
    s2 = jnp.sum(jnp.where(pick2, logits, 0.0), axis=1, keepdims=True)

    g1 = jax.nn.sigmoid(s1)
    g2 = jax.nn.sigmoid(s2)
    denom = g1 + g2 + 1e-6
    comb = (g1 / denom) * pick1.astype(jnp.float32) \
         + (g2 / denom) * pick2.astype(jnp.float32)

    hs = lax.dot_general(xb16, sfc_ref[...], (((1,), (1,)), ((), ())),
                         preferred_element_type=jnp.float32)
    hs = jnp.square(jnp.maximum(hs, 0.0))
    acc = lax.dot_general(hs.astype(jnp.bfloat16), spr_ref[...],
                          (((1,), (1,)), ((), ())),
                          preferred_element_type=jnp.float32)

    for e in range(E):
        h = lax.dot_general(xb16, w1_ref[e], (((1,), (1,)), ((), ())),
                            preferred_element_type=jnp.float32)
        h = jnp.square(jnp.maximum(h, 0.0))
        h = h * comb[:, e:e + 1]
        acc = acc + lax.dot_general(h.astype(jnp.bfloat16), w2_ref[e],
                                    (((1,), (1,)), ((), ())),
                                    preferred_element_type=jnp.float32)

    out_ref[...] = acc


@functools.partial(jax.jit, static_argnames=())
def _run(x2d, gate_w, lb2d, w1_16, w2_16, sfc16, spr16):
    n, d = x2d.shape
    grid = (n // TOK_BLK,)
    full = lambda i: (0, 0)
    full3 = lambda i: (0, 0, 0)
    return pl.pallas_call(
        _dense_body,
        grid=grid,
        in_specs=[
            pl.BlockSpec((TOK_BLK, d), lambda i: (i, 0)),
            pl.BlockSpec((E, d), full),
            pl.BlockSpec((1, E), full),
            pl.BlockSpec((E, 1024, d), full3),
            pl.BlockSpec((E, d, 1024), full3),
            pl.BlockSpec((1024, d), full),
            pl.BlockSpec((d, 1024), full),
        ],
        out_specs=pl.BlockSpec((TOK_BLK, d), lambda i: (i, 0)),
        out_shape=jax.ShapeDtypeStruct((n, d), jnp.float32),
        interpret=_INTERPRET,
    )(x2d, gate_w, lb2d, w1_16, w2_16, sfc16, spr16)


def kernel(x, gate_w, lb_bias, w1, w2, shared_fc, shared_proj):
    b, t, d = x.shape
    x2d = x.reshape(t * b, d)
    out = _run(
        x2d, gate_w, lb_bias.reshape(1, E),
        w1.astype(jnp.bfloat16), w2.astype(jnp.bfloat16),
        shared_fc.astype(jnp.bfloat16), shared_proj.astype(jnp.bfloat16),
    )
    return out.reshape(b, t, d)
